# baseline (speedup 1.0000x reference)
import functools

import jax
import jax.numpy as jnp
from jax import lax
from jax.experimental import pallas as pl
from jax.experimental.pallas import tpu as pltpu
from jax.experimental.pallas import tpu_sc as plsc

_V = 1000000
_K = 64
_B = 16384
_NW = 32
_M = 768
_CH = 128
_NBLK = _V // _CH
_TAIL = _V - _NBLK * _CH
_BASE_BLKS = _NBLK // _NW
_EXTRA = _NBLK - _BASE_BLKS * _NW
_LOOPS = _BASE_BLKS + 1
_RING = 6

_mesh = plsc.VectorSubcoreMesh(core_axis_name="c", subcore_axis_name="s")


def _iota16():
  return lax.iota(jnp.int32, 16)


@functools.partial(
    pl.kernel,
    mesh=_mesh,
    out_type=(
        jax.ShapeDtypeStruct((_NW * _M * _K,), jnp.float32),
        jax.ShapeDtypeStruct((_NW * _M,), jnp.int32),
        jax.ShapeDtypeStruct((_NW * 8,), jnp.int32),
    ),
    compiler_params=pltpu.CompilerParams(needs_layout_passes=False),
    scratch_types=[
        pltpu.VMEM((_B,), jnp.int32),
        pltpu.VMEM((_M + 16,), jnp.int32),
        pltpu.VMEM((_M + 16,), jnp.int32),
        pltpu.VMEM((_M + 16,), jnp.int32),
        pltpu.VMEM((_M + 16,), jnp.int32),
        pltpu.VMEM((_RING, _K, _CH), jnp.float32),
        pltpu.VMEM((_K, _TAIL), jnp.float32),
        pltpu.VMEM((_M * _K,), jnp.float32),
        pltpu.VMEM((_M,), jnp.int32),
        pltpu.VMEM((16,), jnp.int32),
        pltpu.SemaphoreType.DMA,
    ],
)
def _scan_extract(wt_hbm, tail_hbm, uids_hbm, rows_hbm, pos_hbm, cnt_hbm,
                  uids_v, myu_v, myb_v, selu_v, sels_v, blk_v, tail_v, rows_v,
                  posb_v, cntb_v, sem0):
  wid = lax.axis_index("s") * 2 + lax.axis_index("c")
  start_blk = _BASE_BLKS * wid + jnp.minimum(wid, _EXTRA)
  n_blk = _BASE_BLKS + jnp.where(wid < _EXTRA, 1, 0)
  lo = start_blk * _CH
  hi = (start_blk + n_blk) * _CH
  hi = jnp.where(wid == _NW - 1, _V, hi)

  for pb in range(_RING):
    @pl.when(pb < n_blk)
    def _():
      pltpu.async_copy(wt_hbm.at[:, pl.ds((start_blk + pb) * _CH, _CH)],
                       blk_v.at[pb], sem0)

  pltpu.sync_copy(uids_hbm, uids_v)

  big = jnp.full((16,), jnp.int32(0x7FFFFFF0), jnp.int32)
  trash = jnp.full((16,), _B + wid, jnp.int32)
  for c in range((_M + 16) // 16):
    myu_v[pl.ds(16 * c, 16)] = big
    myb_v[pl.ds(16 * c, 16)] = trash

  def filt(t, off):
    vec = uids_v[pl.ds(16 * t, 16)]
    m = jnp.logical_and(vec >= lo, vec < hi)
    offc = jnp.minimum(off, _M)
    plsc.store_compressed(myu_v.at[pl.ds(offc, 16)], vec, mask=m)
    plsc.store_compressed(myb_v.at[pl.ds(offc, 16)], _iota16() + 16 * t,
                          mask=m)
    return off + plsc.all_reduce_population_count(m)[0]

  cnt = lax.fori_loop(0, _B // 16, filt, jnp.int32(0))
  cnt = jnp.minimum(cnt, _M)
  n16 = (cnt + 15) // 16

  def process_block(bref, blo, chw):
    def sel(c, soff):
      uvec = myu_v[pl.ds(16 * c, 16)]
      m2 = jnp.logical_and(uvec >= blo, uvec < blo + chw)
      soffc = jnp.minimum(soff, _M)
      plsc.store_compressed(selu_v.at[pl.ds(soffc, 16)], uvec - blo, mask=m2)
      plsc.store_compressed(sels_v.at[pl.ds(soffc, 16)], _iota16() + 16 * c,
                            mask=m2)
      return soff + plsc.all_reduce_population_count(m2)[0]

    nb = lax.fori_loop(0, n16, sel, jnp.int32(0))
    nb = jnp.minimum(nb, _M)

    def extract(c2, carry):
      ulocv = selu_v[pl.ds(16 * c2, 16)]
      slotv = sels_v[pl.ds(16 * c2, 16)]
      for j in range(16):
        @pl.when(16 * c2 + j < nb)
        def _():
          uloc = ulocv[j]
          slot = jnp.minimum(slotv[j], _M - 1)
          colv = jnp.full((16,), uloc, jnp.int32)
          for r in range(4):
            g = plsc.load_gather(bref, [_iota16() + 16 * r, colv])
            rows_v[pl.ds(slot * _K + 16 * r, 16)] = g
      return carry

    lax.fori_loop(0, (nb + 15) // 16, extract, jnp.int32(0))

  def blk_loop(t, carry):
    cur = lax.rem(t, _RING)

    @pl.when(t < n_blk)
    def _():
      blo = (start_blk + t) * _CH
      pltpu.make_async_copy(wt_hbm.at[:, pl.ds(blo, _CH)], blk_v.at[cur],
                            sem0).wait()
      process_block(blk_v.at[cur], blo, _CH)
      @pl.when(t + _RING < n_blk)
      def _():
        blo3 = (start_blk + t + _RING) * _CH
        pltpu.async_copy(wt_hbm.at[:, pl.ds(blo3, _CH)], blk_v.at[cur], sem0)
    return carry

  lax.fori_loop(0, _LOOPS, blk_loop, jnp.int32(0))

  @pl.when(wid == _NW - 1)
  def _():
    pltpu.sync_copy(tail_hbm, tail_v)
    process_block(tail_v, jnp.int32(_NBLK * _CH), _TAIL)

  for c in range(_M // 16):
    posb_v[pl.ds(16 * c, 16)] = myb_v[pl.ds(16 * c, 16)]
  cntb_v[...] = jnp.full((16,), cnt, jnp.int32)
  pltpu.sync_copy(rows_v, rows_hbm.at[pl.ds(wid * (_M * _K), _M * _K)])
  pltpu.sync_copy(posb_v, pos_hbm.at[pl.ds(wid * _M, _M)])
  pltpu.sync_copy(cntb_v.at[pl.ds(0, 8)], cnt_hbm.at[pl.ds(wid * 8, 8)])


@functools.partial(
    pl.kernel,
    mesh=_mesh,
    out_type=jax.ShapeDtypeStruct((_B + _NW, _K), jnp.float32),
    compiler_params=pltpu.CompilerParams(use_tc_tiling_on_sc=False),
    scratch_types=[
        pltpu.VMEM((_M, _K), jnp.float32),
        pltpu.VMEM((_M // 128, 128), jnp.int32),
        pltpu.VMEM((16,), jnp.int32),
        pltpu.SemaphoreType.DMA,
    ],
)
def _scatter_rows(rows_hbm, pos_hbm, cnt_hbm, out_hbm, rowb_v, posb_v, cntb_v,
                  sem):
  wid = lax.axis_index("s") * 2 + lax.axis_index("c")
  pltpu.sync_copy(rows_hbm.at[pl.ds(wid * _M, _M)], rowb_v)
  pltpu.sync_copy(pos_hbm.at[pl.ds(wid * (_M // 128), _M // 128)], posb_v)
  pltpu.sync_copy(cnt_hbm.at[pl.ds(wid * 8, 8)], cntb_v.at[pl.ds(0, 8)])
  myc = cntb_v[pl.ds(0, 16)][0]
  for j in range(_M // 128):
    @pl.when(128 * j < myc)
    def _():
      pltpu.async_copy(rowb_v.at[pl.ds(128 * j, 128)],
                       out_hbm.at[posb_v.at[j]], sem).wait()


def _softmax_matmul_body(g_ref, p_ref, o_ref):
  w = g_ref[...]
  m = jnp.max(w, axis=-1, keepdims=True)
  e = jnp.exp(w - m)
  s = jnp.sum(e, axis=-1, keepdims=True)
  o_ref[...] = jnp.dot(e / s, p_ref[...], preferred_element_type=jnp.float32)


def _softmax_matmul(g, P):
  BLK = 2048
  return pl.pallas_call(
      _softmax_matmul_body,
      grid=(_B // BLK,),
      in_specs=[
          pl.BlockSpec((BLK, _K), lambda i: (i, 0)),
          pl.BlockSpec((_K, _K), lambda i: (0, 0)),
      ],
      out_specs=pl.BlockSpec((BLK, _K), lambda i: (i, 0)),
      out_shape=jax.ShapeDtypeStruct((_B, _K), jnp.float32),
  )(g, P)


def kernel(W, P, u_ids):
  tail = W.T[:, _NBLK * _CH:]
  rows_flat, pos, cnt = _scan_extract(W.T, tail, u_ids.astype(jnp.int32))
  rows2 = rows_flat.reshape(_NW * _M, _K)
  pos2 = pos.reshape(_NW * (_M // 128), 128)
  g = _scatter_rows(rows2, pos2, cnt)
  return _softmax_matmul(g[:_B], P)


# speedup vs baseline: 1.4995x; 1.4995x over previous
import functools

import jax
import jax.numpy as jnp
from jax import lax
from jax.experimental import pallas as pl
from jax.experimental.pallas import tpu as pltpu
from jax.experimental.pallas import tpu_sc as plsc

_V = 1000000
_K = 64
_B = 16384
_NW = 32
_M = 768
_CH = 256
_NBLK = _V // _CH
_TAIL = _V - _NBLK * _CH
_BASE_BLKS = _NBLK // _NW
_EXTRA = _NBLK - _BASE_BLKS * _NW
_LOOPS = _BASE_BLKS + 1
_RING = 3

_mesh = plsc.VectorSubcoreMesh(core_axis_name="c", subcore_axis_name="s")


def _iota16():
  return lax.iota(jnp.int32, 16)


@functools.partial(
    pl.kernel,
    mesh=_mesh,
    out_type=jax.ShapeDtypeStruct((_B * _K,), jnp.float32),
    compiler_params=pltpu.CompilerParams(needs_layout_passes=False),
    scratch_types=[
        pltpu.VMEM((_B,), jnp.int32),
        pltpu.VMEM((_M + 16,), jnp.int32),
        pltpu.VMEM((_M + 16,), jnp.int32),
        pltpu.VMEM((_M + 16,), jnp.int32),
        pltpu.VMEM((_M + 16,), jnp.int32),
        pltpu.VMEM((_RING, _K, _CH), jnp.float32),
        pltpu.VMEM((_K, _TAIL), jnp.float32),
        pltpu.VMEM((_M * _K,), jnp.float32),
        pltpu.SemaphoreType.DMA,
        pltpu.SemaphoreType.DMA,
    ],
)
def _scan_gather(wt_hbm, tail_hbm, uids_hbm, out_hbm,
                 uids_v, myu_v, myb_v, selu_v, sels_v, blk_v, tail_v, rows_v,
                 sem0, sem1):
  wid = lax.axis_index("s") * 2 + lax.axis_index("c")
  start_blk = _BASE_BLKS * wid + jnp.minimum(wid, _EXTRA)
  n_blk = _BASE_BLKS + jnp.where(wid < _EXTRA, 1, 0)
  lo = start_blk * _CH
  hi = (start_blk + n_blk) * _CH
  hi = jnp.where(wid == _NW - 1, _V, hi)

  for pb in range(_RING):
    @pl.when(pb < n_blk)
    def _():
      pltpu.async_copy(wt_hbm.at[:, pl.ds((start_blk + pb) * _CH, _CH)],
                       blk_v.at[pb], sem0)

  pltpu.sync_copy(uids_hbm, uids_v)

  big = jnp.full((16,), jnp.int32(0x7FFFFFF0), jnp.int32)
  for c in range((_M + 16) // 16):
    myu_v[pl.ds(16 * c, 16)] = big

  def filt(t, off):
    vec = uids_v[pl.ds(16 * t, 16)]
    m = jnp.logical_and(vec >= lo, vec < hi)
    offc = jnp.minimum(off, _M)
    plsc.store_compressed(myu_v.at[pl.ds(offc, 16)], vec, mask=m)
    plsc.store_compressed(myb_v.at[pl.ds(offc, 16)], _iota16() + 16 * t,
                          mask=m)
    return off + plsc.all_reduce_population_count(m)[0]

  cnt = lax.fori_loop(0, _B // 16, filt, jnp.int32(0))
  cnt = jnp.minimum(cnt, _M)
  n16 = (cnt + 15) // 16

  def process_block(bref, blo, chw):
    def sel(c, soff):
      uvec = myu_v[pl.ds(16 * c, 16)]
      m2 = jnp.logical_and(uvec >= blo, uvec < blo + chw)
      soffc = jnp.minimum(soff, _M)
      plsc.store_compressed(selu_v.at[pl.ds(soffc, 16)], uvec - blo, mask=m2)
      plsc.store_compressed(sels_v.at[pl.ds(soffc, 16)], _iota16() + 16 * c,
                            mask=m2)
      return soff + plsc.all_reduce_population_count(m2)[0]

    nb = lax.fori_loop(0, n16, sel, jnp.int32(0))
    nb = jnp.minimum(nb, _M)

    def extract(c2, carry):
      ulocv = selu_v[pl.ds(16 * c2, 16)]
      slotv = sels_v[pl.ds(16 * c2, 16)]
      for j in range(16):
        @pl.when(16 * c2 + j < nb)
        def _():
          uloc = ulocv[j]
          slot = jnp.minimum(slotv[j], _M - 1)
          colv = jnp.full((16,), uloc, jnp.int32)
          for r in range(4):
            g = plsc.load_gather(bref, [_iota16() + 16 * r, colv])
            rows_v[pl.ds(slot * _K + 16 * r, 16)] = g
      return carry

    lax.fori_loop(0, (nb + 15) // 16, extract, jnp.int32(0))

  def blk_loop(t, carry):
    cur = lax.rem(t, _RING)

    @pl.when(t < n_blk)
    def _():
      blo = (start_blk + t) * _CH
      pltpu.make_async_copy(wt_hbm.at[:, pl.ds(blo, _CH)], blk_v.at[cur],
                            sem0).wait()
      process_block(blk_v.at[cur], blo, _CH)
      @pl.when(t + _RING < n_blk)
      def _():
        blo3 = (start_blk + t + _RING) * _CH
        pltpu.async_copy(wt_hbm.at[:, pl.ds(blo3, _CH)], blk_v.at[cur], sem0)
    return carry

  lax.fori_loop(0, _LOOPS, blk_loop, jnp.int32(0))

  @pl.when(wid == _NW - 1)
  def _():
    pltpu.sync_copy(tail_hbm, tail_v)
    process_block(tail_v, jnp.int32(_NBLK * _CH), _TAIL)

  def scat(c, carry):
    posv = myb_v[pl.ds(16 * c, 16)]
    for j in range(16):
      @pl.when(16 * c + j < cnt)
      def _():
        pltpu.async_copy(rows_v.at[pl.ds((16 * c + j) * _K, _K)],
                         out_hbm.at[pl.ds(posv[j] * _K, _K)], sem1)
    for j in range(16):
      @pl.when(16 * c + j < cnt)
      def _():
        pltpu.make_async_copy(rows_v.at[pl.ds((16 * c + j) * _K, _K)],
                              out_hbm.at[pl.ds(posv[j] * _K, _K)],
                              sem1).wait()
    return carry

  lax.fori_loop(0, n16, scat, jnp.int32(0))


def _softmax_matmul_body(g_ref, p_ref, o_ref):
  w = g_ref[...]
  m = jnp.max(w, axis=-1, keepdims=True)
  e = jnp.exp(w - m)
  s = jnp.sum(e, axis=-1, keepdims=True)
  o_ref[...] = jnp.dot(e / s, p_ref[...], preferred_element_type=jnp.float32)


def _softmax_matmul(g, P):
  BLK = 2048
  return pl.pallas_call(
      _softmax_matmul_body,
      grid=(_B // BLK,),
      in_specs=[
          pl.BlockSpec((BLK, _K), lambda i: (i, 0)),
          pl.BlockSpec((_K, _K), lambda i: (0, 0)),
      ],
      out_specs=pl.BlockSpec((BLK, _K), lambda i: (i, 0)),
      out_shape=jax.ShapeDtypeStruct((_B, _K), jnp.float32),
  )(g, P)


def kernel(W, P, u_ids):
  tail = W.T[:, _NBLK * _CH:]
  g_flat = _scan_gather(W.T, tail, u_ids.astype(jnp.int32))
  return _softmax_matmul(g_flat.reshape(_B, _K), P)


# speedup vs baseline: 1.5851x; 1.0570x over previous
import functools

import jax
import jax.numpy as jnp
from jax import lax
from jax.experimental import pallas as pl
from jax.experimental.pallas import tpu as pltpu
from jax.experimental.pallas import tpu_sc as plsc

_V = 1000000
_K = 64
_B = 16384
_NW = 32
_M = 768
_CH = 256
_NBLK = _V // _CH
_TAIL = _V - _NBLK * _CH
_BASE_BLKS = _NBLK // _NW
_EXTRA = _NBLK - _BASE_BLKS * _NW
_LOOPS = _BASE_BLKS + 1
_RING = 3

_mesh = plsc.VectorSubcoreMesh(core_axis_name="c", subcore_axis_name="s")


def _iota16():
  return lax.iota(jnp.int32, 16)


@functools.partial(
    pl.kernel,
    mesh=_mesh,
    out_type=jax.ShapeDtypeStruct((_B * _K,), jnp.float32),
    compiler_params=pltpu.CompilerParams(needs_layout_passes=False),
    scratch_types=[
        pltpu.VMEM((_B,), jnp.int32),
        pltpu.VMEM((_M + 16,), jnp.int32),
        pltpu.VMEM((_M + 16,), jnp.int32),
        pltpu.VMEM((_M + 16,), jnp.int32),
        pltpu.VMEM((_M + 16,), jnp.int32),
        pltpu.VMEM((_RING, _K, _CH), jnp.float32),
        pltpu.VMEM((_K, _TAIL), jnp.float32),
        pltpu.VMEM((_M * _K,), jnp.float32),
        pltpu.SemaphoreType.DMA,
        pltpu.SemaphoreType.DMA,
    ],
)
def _scan_gather(wt_hbm, tail_hbm, uids_hbm, out_hbm,
                 uids_v, myu_v, myb_v, selu_v, sels_v, blk_v, tail_v, rows_v,
                 sem0, sem1):
  wid = lax.axis_index("s") * 2 + lax.axis_index("c")
  start_blk = _BASE_BLKS * wid + jnp.minimum(wid, _EXTRA)
  n_blk = _BASE_BLKS + jnp.where(wid < _EXTRA, 1, 0)
  lo = start_blk * _CH
  hi = (start_blk + n_blk) * _CH
  hi = jnp.where(wid == _NW - 1, _V, hi)

  for pb in range(_RING):
    @pl.when(pb < n_blk)
    def _():
      pltpu.async_copy(wt_hbm.at[:, pl.ds((start_blk + pb) * _CH, _CH)],
                       blk_v.at[pb], sem0)

  pltpu.sync_copy(uids_hbm, uids_v)

  big = jnp.full((16,), jnp.int32(0x7FFFFFF0), jnp.int32)
  for c in range((_M + 16) // 16):
    myu_v[pl.ds(16 * c, 16)] = big

  def filt(t, off):
    vec = uids_v[pl.ds(16 * t, 16)]
    m = jnp.logical_and(vec >= lo, vec < hi)
    offc = jnp.minimum(off, _M)
    plsc.store_compressed(myu_v.at[pl.ds(offc, 16)], vec, mask=m)
    plsc.store_compressed(myb_v.at[pl.ds(offc, 16)], _iota16() + 16 * t,
                          mask=m)
    return off + plsc.all_reduce_population_count(m)[0]

  cnt = lax.fori_loop(0, _B // 16, filt, jnp.int32(0))
  cnt = jnp.minimum(cnt, _M)
  n16 = (cnt + 15) // 16

  def process_block(bref, blo, chw):
    def sel(c, soff):
      uvec = myu_v[pl.ds(16 * c, 16)]
      m2 = jnp.logical_and(uvec >= blo, uvec < blo + chw)
      soffc = jnp.minimum(soff, _M)
      plsc.store_compressed(selu_v.at[pl.ds(soffc, 16)], uvec - blo, mask=m2)
      plsc.store_compressed(sels_v.at[pl.ds(soffc, 16)], _iota16() + 16 * c,
                            mask=m2)
      return soff + plsc.all_reduce_population_count(m2)[0]

    nb = lax.fori_loop(0, n16, sel, jnp.int32(0))
    nb = jnp.minimum(nb, _M)

    def extract(c2, carry):
      ulocv = selu_v[pl.ds(16 * c2, 16)]
      slotv = sels_v[pl.ds(16 * c2, 16)]
      for j in range(16):
        @pl.when(16 * c2 + j < nb)
        def _():
          uloc = ulocv[j]
          slot = jnp.minimum(slotv[j], _M - 1)
          colv = jnp.full((16,), uloc, jnp.int32)
          for r in range(4):
            g = plsc.load_gather(bref, [_iota16() + 16 * r, colv])
            rows_v[pl.ds(slot * _K + 16 * r, 16)] = g
      return carry

    lax.fori_loop(0, (nb + 15) // 16, extract, jnp.int32(0))

  def blk_loop(t, carry):
    cur = lax.rem(t, _RING)

    @pl.when(t < n_blk)
    def _():
      blo = (start_blk + t) * _CH
      pltpu.make_async_copy(wt_hbm.at[:, pl.ds(blo, _CH)], blk_v.at[cur],
                            sem0).wait()
      process_block(blk_v.at[cur], blo, _CH)
      @pl.when(t + _RING < n_blk)
      def _():
        blo3 = (start_blk + t + _RING) * _CH
        pltpu.async_copy(wt_hbm.at[:, pl.ds(blo3, _CH)], blk_v.at[cur], sem0)
    return carry

  lax.fori_loop(0, _LOOPS, blk_loop, jnp.int32(0))

  @pl.when(wid == _NW - 1)
  def _():
    pltpu.sync_copy(tail_hbm, tail_v)
    process_block(tail_v, jnp.int32(_NBLK * _CH), _TAIL)

  def fire(c):
    posv = myb_v[pl.ds(16 * c, 16)]
    for j in range(16):
      @pl.when(16 * c + j < cnt)
      def _():
        pltpu.async_copy(rows_v.at[pl.ds((16 * c + j) * _K, _K)],
                         out_hbm.at[pl.ds(posv[j] * _K, _K)], sem1)

  def drain(c):
    posv = myb_v[pl.ds(16 * c, 16)]
    for j in range(16):
      @pl.when(16 * c + j < cnt)
      def _():
        pltpu.make_async_copy(rows_v.at[pl.ds((16 * c + j) * _K, _K)],
                              out_hbm.at[pl.ds(posv[j] * _K, _K)],
                              sem1).wait()

  @pl.when(n16 > 0)
  def _():
    fire(jnp.int32(0))

  def scat(c, carry):
    @pl.when(c + 1 < n16)
    def _():
      fire(c + 1)
    drain(c)
    return carry

  lax.fori_loop(0, n16, scat, jnp.int32(0))


def _softmax_matmul_body(g_ref, p_ref, o_ref):
  w = g_ref[...]
  m = jnp.max(w, axis=-1, keepdims=True)
  e = jnp.exp(w - m)
  s = jnp.sum(e, axis=-1, keepdims=True)
  probs = e / s
  o_ref[...] = jax.lax.dot_general(
      p_ref[...], probs, (((0,), (1,)), ((), ())),
      preferred_element_type=jnp.float32)


def _softmax_matmul_t(g, P):
  BLK = 2048
  return pl.pallas_call(
      _softmax_matmul_body,
      grid=(_B // BLK,),
      in_specs=[
          pl.BlockSpec((BLK, _K), lambda i: (i, 0)),
          pl.BlockSpec((_K, _K), lambda i: (0, 0)),
      ],
      out_specs=pl.BlockSpec((_K, BLK), lambda i: (0, i)),
      out_shape=jax.ShapeDtypeStruct((_K, _B), jnp.float32),
  )(g, P)


def kernel(W, P, u_ids):
  tail = W.T[:, _NBLK * _CH:]
  g_flat = _scan_gather(W.T, tail, u_ids.astype(jnp.int32))
  return _softmax_matmul_t(g_flat.reshape(_B, _K), P).T
